# Initial kernel scaffold; baseline (speedup 1.0000x reference)
import jax
import jax.numpy as jnp
from jax.experimental import pallas as pl


def kernel(x, edge_index, bn1_gamma, bn1_beta, W1, b1, W2, b2, W3, b3, W4, b4, bn2_gamma, bn2_beta, Wd, bd):
    raise NotImplementedError("write your pallas kernel here")


# speedup vs baseline: 7.3767x; 7.3767x over previous
import functools

import jax
import jax.numpy as jnp
from jax import lax
from jax.experimental import pallas as pl
from jax.experimental.pallas import tpu as pltpu
from jax.experimental.pallas import tpu_sc as plsc

N = 10000
E = 320000
D_IN = 128
D_H = 256
D_OUT = 5
DHALF = D_H // 2

NC = 2
NS = 16
CHUNK = 128
E2 = E + N
C_PER_TEC = -(-E2 // (NS * CHUNK))
EDGES_PER_TEC = C_PER_TEC * CHUNK
E_PAD = EDGES_PER_TEC * NS
N_ACC = 10240
ZROWS = N_ACC // NS
NZ = ZROWS // CHUNK
OROWS = (N // NS) // 8 * 8
OTAIL = N - OROWS * NS

_MESH = plsc.VectorSubcoreMesh(
    core_axis_name="c", subcore_axis_name="s", num_cores=NC, num_subcores=NS
)


DEGW = 128


@functools.partial(
    pl.kernel,
    out_type=jax.ShapeDtypeStruct((N, DEGW), jnp.float32),
    mesh=_MESH,
    scratch_types=[
        pltpu.VMEM((CHUNK,), jnp.int32),
        pltpu.VMEM((CHUNK, DEGW), jnp.float32),
        pltpu.VMEM_SHARED((N_ACC, DEGW), jnp.float32),
    ],
)
def _sc_degree(dst_hbm, z16_hbm, o16_hbm, deg_hbm, idx_v, val_v, acc_s):
    cid = lax.axis_index("c")
    sid = lax.axis_index("s")

    @pl.when(cid == 0)
    def _():
        pltpu.sync_copy(z16_hbm, val_v)
        for j in range(NZ):
            pltpu.sync_copy(val_v, acc_s.at[pl.ds(sid * ZROWS + j * CHUNK, CHUNK)])
        plsc.subcore_barrier()

        pltpu.sync_copy(o16_hbm, val_v)
        base = sid * EDGES_PER_TEC

        def body(i, carry):
            pltpu.sync_copy(dst_hbm.at[pl.ds(base + i * CHUNK, CHUNK)], idx_v)
            pltpu.sync_copy(val_v, acc_s.at[idx_v], add=True)
            return carry

        lax.fori_loop(0, C_PER_TEC, body, 0)
        plsc.subcore_barrier()
        pltpu.sync_copy(
            acc_s.at[pl.ds(sid * OROWS, OROWS)], deg_hbm.at[pl.ds(sid * OROWS, OROWS)]
        )

        @pl.when(sid == 0)
        def _():
            pltpu.sync_copy(
                acc_s.at[pl.ds(OROWS * NS, OTAIL)],
                deg_hbm.at[pl.ds(OROWS * NS, OTAIL)],
            )


@functools.partial(
    pl.kernel,
    out_type=[jax.ShapeDtypeStruct((N, DHALF), jnp.float32)] * 2,
    mesh=_MESH,
    scratch_types=[
        pltpu.VMEM((CHUNK,), jnp.int32),
        pltpu.VMEM((CHUNK,), jnp.int32),
        pltpu.VMEM((CHUNK, DHALF), jnp.float32),
        pltpu.VMEM((CHUNK, DHALF), jnp.float32),
        pltpu.VMEM_SHARED((N_ACC, DHALF), jnp.float32),
        pltpu.SemaphoreType.DMA,
    ],
)
def _sc_aggregate(
    src_hbm, dst_hbm, ua_hbm, ub_hbm, zrow_hbm, oa_hbm, ob_hbm,
    sidx, didx, rows, zbuf, acc_s, sem,
):
    cid = lax.axis_index("c")
    sid = lax.axis_index("s")
    base = sid * EDGES_PER_TEC

    pltpu.sync_copy(zrow_hbm, zbuf)
    for j in range(NZ):
        pltpu.sync_copy(zbuf, acc_s.at[pl.ds(sid * ZROWS + j * CHUNK, CHUNK)])
    plsc.subcore_barrier()

    def process(h_hbm, o_hbm):
        def body(i, carry):
            pltpu.sync_copy(src_hbm.at[pl.ds(base + i * CHUNK, CHUNK)], sidx)
            pltpu.sync_copy(dst_hbm.at[pl.ds(base + i * CHUNK, CHUNK)], didx)
            pltpu.async_copy(h_hbm.at[sidx], rows, sem).wait()
            pltpu.sync_copy(rows, acc_s.at[didx], add=True)
            return carry

        lax.fori_loop(0, C_PER_TEC, body, 0)
        plsc.subcore_barrier()
        pltpu.sync_copy(
            acc_s.at[pl.ds(sid * OROWS, OROWS)], o_hbm.at[pl.ds(sid * OROWS, OROWS)]
        )

        @pl.when(sid == 0)
        def _():
            pltpu.sync_copy(
                acc_s.at[pl.ds(OROWS * NS, OTAIL)],
                o_hbm.at[pl.ds(OROWS * NS, OTAIL)],
            )

    @pl.when(cid == 0)
    def _():
        process(ua_hbm, oa_hbm)

    @pl.when(cid == 1)
    def _():
        process(ub_hbm, ob_hbm)


def _dinv_col(deg_ref):
    return lax.rsqrt(jnp.maximum(deg_ref[...][:, 0:1], 1.0))


def _tc_pre_body(x_ref, g_ref, b_ref, w_ref, deg_ref, ua_ref, ub_ref):
    xv = x_ref[...]
    mean = jnp.mean(xv, axis=0, keepdims=True)
    var = jnp.mean((xv - mean) * (xv - mean), axis=0, keepdims=True)
    h = (xv - mean) / jnp.sqrt(var + 1e-5) * g_ref[...] + b_ref[...]
    dinv = _dinv_col(deg_ref)
    u = jnp.dot(h, w_ref[...], preferred_element_type=jnp.float32) * dinv
    ua_ref[...] = u[:, :DHALF]
    ub_ref[...] = u[:, DHALF:]


_tc_pre = pl.pallas_call(
    _tc_pre_body,
    out_shape=[jax.ShapeDtypeStruct((N, DHALF), jnp.float32)] * 2,
)


def _tc_mid_body(ga_ref, gb_ref, deg_ref, b_ref, w_ref, ua_ref, ub_ref):
    dinv = _dinv_col(deg_ref)
    bv = b_ref[...]
    ha = ga_ref[...] * dinv + bv[:, :DHALF]
    hb = gb_ref[...] * dinv + bv[:, DHALF:]
    wv = w_ref[...]
    u = (
        jnp.dot(ha, wv[:DHALF, :], preferred_element_type=jnp.float32)
        + jnp.dot(hb, wv[DHALF:, :], preferred_element_type=jnp.float32)
    ) * dinv
    ua_ref[...] = u[:, :DHALF]
    ub_ref[...] = u[:, DHALF:]


_tc_mid = pl.pallas_call(
    _tc_mid_body,
    out_shape=[jax.ShapeDtypeStruct((N, DHALF), jnp.float32)] * 2,
)


def _tc_post_body(
    ga_ref, gb_ref, deg_ref, b_ref, g2_ref, b2_ref, wd_ref, bd_ref, out_ref
):
    dinv = _dinv_col(deg_ref)
    bv = b_ref[...]
    g2 = g2_ref[...]
    b2 = b2_ref[...]

    def bn_half(h, sl):
        mean = jnp.mean(h, axis=0, keepdims=True)
        var = jnp.mean((h - mean) * (h - mean), axis=0, keepdims=True)
        return (h - mean) / jnp.sqrt(var + 1e-5) * g2[:, sl] + b2[:, sl]

    ha = bn_half(ga_ref[...] * dinv + bv[:, :DHALF], slice(0, DHALF))
    hb = bn_half(gb_ref[...] * dinv + bv[:, DHALF:], slice(DHALF, D_H))
    wd = wd_ref[...]
    logits = (
        jnp.dot(ha, wd[:DHALF, :], preferred_element_type=jnp.float32)
        + jnp.dot(hb, wd[DHALF:, :], preferred_element_type=jnp.float32)
        + bd_ref[...]
    )
    out_ref[...] = jax.nn.sigmoid(logits)


_tc_post = pl.pallas_call(
    _tc_post_body,
    out_shape=jax.ShapeDtypeStruct((N, D_OUT), jnp.float32),
)


def kernel(
    x, edge_index, bn1_gamma, bn1_beta, W1, b1, W2, b2, W3, b3, W4, b4,
    bn2_gamma, bn2_beta, Wd, bd,
):
    loop = jnp.arange(N, dtype=jnp.int32)
    src = jnp.concatenate(
        [edge_index[0].astype(jnp.int32), loop,
         jnp.zeros((E_PAD - E2,), jnp.int32)]
    )
    dst = jnp.concatenate(
        [edge_index[1].astype(jnp.int32), loop,
         jnp.full((E_PAD - E2,), N, jnp.int32)]
    )

    z16 = jnp.zeros((CHUNK, DEGW), jnp.float32)
    o16 = jnp.ones((CHUNK, DEGW), jnp.float32)
    zrow = jnp.zeros((CHUNK, DHALF), jnp.float32)

    deg16 = _sc_degree(dst, z16, o16)

    ua, ub = _tc_pre(
        x, bn1_gamma.reshape(1, D_IN), bn1_beta.reshape(1, D_IN), W1, deg16
    )
    ga, gb = _sc_aggregate(src, dst, ua, ub, zrow)
    ua, ub = _tc_mid(ga, gb, deg16, b1.reshape(1, D_H), W2)
    ga, gb = _sc_aggregate(src, dst, ua, ub, zrow)
    ua, ub = _tc_mid(ga, gb, deg16, b2.reshape(1, D_H), W3)
    ga, gb = _sc_aggregate(src, dst, ua, ub, zrow)
    ua, ub = _tc_mid(ga, gb, deg16, b3.reshape(1, D_H), W4)
    ga, gb = _sc_aggregate(src, dst, ua, ub, zrow)

    return _tc_post(
        ga, gb, deg16, b4.reshape(1, D_H),
        bn2_gamma.reshape(1, D_H), bn2_beta.reshape(1, D_H), Wd,
        bd.reshape(1, D_OUT),
    )
